# baseline (speedup 1.0000x reference)
import functools

import jax
import jax.numpy as jnp
from jax.experimental import pallas as pl

_B = 4
_LATENT = 128
_N0 = 512
_CHANNELS = [32, 16, 8]
_NODES = [1024, 2048, 4096]


def _split_dot(w_bf16, x_f32):
    x_hi = x_f32.astype(jnp.bfloat16)
    x_lo = (x_f32 - x_hi.astype(jnp.float32)).astype(jnp.bfloat16)
    acc = jnp.dot(w_bf16, x_hi, preferred_element_type=jnp.float32)
    return acc + jnp.dot(w_bf16, x_lo, preferred_element_type=jnp.float32)


def _fc_kernel(lat_ref, w_ref, b_ref, o_ref):
    lat = lat_ref[...]
    lat_hi = lat.astype(jnp.bfloat16)
    lat_lo = (lat - lat_hi.astype(jnp.float32)).astype(jnp.bfloat16)
    w = w_ref[...].astype(jnp.bfloat16)
    acc = jnp.dot(lat_hi, w, preferred_element_type=jnp.float32)
    acc = acc + jnp.dot(lat_lo, w, preferred_element_type=jnp.float32)
    o_ref[...] = acc + b_ref[0]


def _u_kernel(u_ref, x_ref, ub_ref, o_ref):
    y = _split_dot(u_ref[...].astype(jnp.bfloat16), x_ref[...])
    o_ref[...] = y + jnp.tile(ub_ref[...], (1, _B))


def _l_kernel(l_ref, y_ref, w_ref, b_ref, o_ref, *, c_in, c_out):
    z = _split_dot(l_ref[...].astype(jnp.bfloat16), y_ref[...])
    w = w_ref[...]
    wt = jnp.tile(w, (_B, _B))
    rows = jax.lax.broadcasted_iota(jnp.int32, wt.shape, 0) // c_in
    cols = jax.lax.broadcasted_iota(jnp.int32, wt.shape, 1) // c_out
    wbd = jnp.where(rows == cols, wt, 0.0)
    z_hi = z.astype(jnp.bfloat16)
    z_lo = (z - z_hi.astype(jnp.float32)).astype(jnp.bfloat16)
    wbd_bf = wbd.astype(jnp.bfloat16)
    h = jnp.dot(z_hi, wbd_bf, preferred_element_type=jnp.float32)
    h = h + jnp.dot(z_lo, wbd_bf, preferred_element_type=jnp.float32)
    o_ref[...] = jnp.maximum(h + jnp.tile(b_ref[...], (1, _B)), 0.0)


def _fc(latents, fc_W, fc_b, *, tc=2048):
    n_total = fc_W.shape[1]
    grid = n_total // tc
    b3 = fc_b.reshape(grid, 1, tc)
    return pl.pallas_call(
        _fc_kernel,
        grid=(grid,),
        in_specs=[
            pl.BlockSpec((_B, _LATENT), lambda i: (0, 0)),
            pl.BlockSpec((_LATENT, tc), lambda i: (0, i)),
            pl.BlockSpec((1, 1, tc), lambda i: (i, 0, 0)),
        ],
        out_specs=pl.BlockSpec((_B, tc), lambda i: (0, i)),
        out_shape=jax.ShapeDtypeStruct((_B, n_total), jnp.float32),
    )(latents, fc_W, b3)


def _u_mm(U, X, Ub, *, tm=256):
    m, k = U.shape
    n = X.shape[1]
    c_in = Ub.shape[1]
    return pl.pallas_call(
        _u_kernel,
        grid=(m // tm,),
        in_specs=[
            pl.BlockSpec((tm, k), lambda i: (i, 0)),
            pl.BlockSpec((k, n), lambda i: (0, 0)),
            pl.BlockSpec((tm, c_in), lambda i: (i, 0)),
        ],
        out_specs=pl.BlockSpec((tm, n), lambda i: (i, 0)),
        out_shape=jax.ShapeDtypeStruct((m, n), jnp.float32),
    )(U, X, Ub)


def _l_mm(L, Y, W, b, *, tm=256):
    m, k = L.shape
    n = Y.shape[1]
    c_in, c_out = W.shape
    kern = functools.partial(_l_kernel, c_in=c_in, c_out=c_out)
    return pl.pallas_call(
        kern,
        grid=(m // tm,),
        in_specs=[
            pl.BlockSpec((tm, k), lambda i: (i, 0)),
            pl.BlockSpec((k, n), lambda i: (0, 0)),
            pl.BlockSpec((c_in, c_out), lambda i: (0, 0)),
            pl.BlockSpec((1, c_out), lambda i: (0, 0)),
        ],
        out_specs=pl.BlockSpec((tm, _B * c_out), lambda i: (i, 0)),
        out_shape=jax.ShapeDtypeStruct((m, _B * c_out), jnp.float32),
    )(L, Y, W, b.reshape(1, c_out))


def kernel(latents, fc_W, fc_b, U0, Ub0, L0, W0, b0, U1, Ub1, L1, W1, b1, U2, Ub2, L2, W2, b2):
    feats = _fc(latents, fc_W, fc_b)
    x = feats.reshape(_B, _N0, _CHANNELS[0]).transpose(1, 0, 2).reshape(_N0, _B * _CHANNELS[0])
    for U, Ub, L, W, b in ((U0, Ub0, L0, W0, b0), (U1, Ub1, L1, W1, b1), (U2, Ub2, L2, W2, b2)):
        y = _u_mm(U, x, Ub)
        x = _l_mm(L, y, W, b)
    n_last = _NODES[-1]
    c_last = _CHANNELS[-1]
    return x.reshape(n_last, _B, c_last).transpose(1, 0, 2)


# speedup vs baseline: 1.1315x; 1.1315x over previous
import jax
import jax.numpy as jnp
from jax.experimental import pallas as pl
from jax.experimental.pallas import tpu as pltpu

_B = 4
_LATENT = 128
_N0 = 512
_CHANNELS = [32, 16, 8]
_NODES = [1024, 2048, 4096]
_CIN = [32, 32, 16]
_COUT = [32, 16, 8]
_TM = 512


def _split_dot(w_bf16, x_f32):
    x_hi = x_f32.astype(jnp.bfloat16)
    x_lo = (x_f32 - x_hi.astype(jnp.float32)).astype(jnp.bfloat16)
    acc = jnp.dot(w_bf16, x_hi, preferred_element_type=jnp.float32)
    return acc + jnp.dot(w_bf16, x_lo, preferred_element_type=jnp.float32)


def _block_diag(w, c_in, c_out):
    wt = jnp.tile(w, (_B, _B))
    rows = jax.lax.broadcasted_iota(jnp.int32, wt.shape, 0) // c_in
    cols = jax.lax.broadcasted_iota(jnp.int32, wt.shape, 1) // c_out
    return jnp.where(rows == cols, wt, 0.0)


def _fc_kernel(lat_ref, w_ref, b_ref, o_ref):
    lat = lat_ref[...]
    lat_hi = lat.astype(jnp.bfloat16)
    lat_lo = (lat - lat_hi.astype(jnp.float32)).astype(jnp.bfloat16)
    w = w_ref[...].astype(jnp.bfloat16)
    acc = jnp.dot(lat_hi, w, preferred_element_type=jnp.float32)
    acc = acc + jnp.dot(lat_lo, w, preferred_element_type=jnp.float32)
    o_ref[...] = acc + b_ref[0]


def _fc(latents, fc_W, fc_b, *, tc=2048):
    n_total = fc_W.shape[1]
    grid = n_total // tc
    b3 = fc_b.reshape(grid, 1, tc)
    return pl.pallas_call(
        _fc_kernel,
        grid=(grid,),
        in_specs=[
            pl.BlockSpec((_B, _LATENT), lambda i: (0, 0)),
            pl.BlockSpec((_LATENT, tc), lambda i: (0, i)),
            pl.BlockSpec((1, 1, tc), lambda i: (i, 0, 0)),
        ],
        out_specs=pl.BlockSpec((_B, tc), lambda i: (0, i)),
        out_shape=jax.ShapeDtypeStruct((_B, n_total), jnp.float32),
    )(latents, fc_W, b3)


def _mega_body(x0_ref, ub0_ref, w0_ref, b0_ref, ub1_ref, w1_ref, b1_ref,
               ub2_ref, w2_ref, b2_ref,
               u0_hbm, l0_hbm, u1_hbm, l1_hbm, u2_hbm, l2_hbm,
               out_ref, buf_b, buf_c, buf_d, buf_e,
               y0, x1, y1, x2, y2, sems):
    chain = (x0_ref, y0, x1, y1, x2, y2, out_ref)
    hbm_refs = (u0_hbm, l0_hbm, u1_hbm, l1_hbm, u2_hbm, l2_hbm)
    bufs = (buf_b, buf_c, buf_c, buf_d, buf_d, buf_e)
    ub_refs = (ub0_ref, ub1_ref, ub2_ref)
    w_refs = (w0_ref, w1_ref, w2_ref)
    b_refs = (b0_ref, b1_ref, b2_ref)

    def u_compute(buf, slot, i, x_in, y_out, ub_ref):
        def run():
            w = buf[slot].astype(jnp.bfloat16)
            y = _split_dot(w, x_in[...])
            ub = ub_ref[i * _TM:(i + 1) * _TM, :]
            y_out[i * _TM:(i + 1) * _TM, :] = y + jnp.tile(ub, (1, _B))
        return run

    def l_compute(buf, slot, i, y_in, x_out, w_ref, b_ref, c_in, c_out):
        def run():
            w = buf[slot].astype(jnp.bfloat16)
            z = _split_dot(w, y_in[...])
            wbd = _block_diag(w_ref[...], c_in, c_out).astype(jnp.bfloat16)
            z_hi = z.astype(jnp.bfloat16)
            z_lo = (z - z_hi.astype(jnp.float32)).astype(jnp.bfloat16)
            h = jnp.dot(z_hi, wbd, preferred_element_type=jnp.float32)
            h = h + jnp.dot(z_lo, wbd, preferred_element_type=jnp.float32)
            h = h + jnp.tile(b_ref[...], (1, _B))
            x_out[i * _TM:(i + 1) * _TM, :] = jnp.maximum(h, 0.0)
        return run

    copies = []
    computes = []
    g = 0
    for p in range(6):
        lvl = p // 2
        for i in range(_NODES[lvl] // _TM):
            slot = g % 2
            copies.append(pltpu.make_async_copy(
                hbm_refs[p].at[i * _TM:(i + 1) * _TM, :],
                bufs[p].at[slot], sems.at[slot]))
            if p % 2 == 0:
                computes.append(u_compute(bufs[p], slot, i, chain[p],
                                          chain[p + 1], ub_refs[lvl]))
            else:
                computes.append(l_compute(bufs[p], slot, i, chain[p],
                                          chain[p + 1], w_refs[lvl],
                                          b_refs[lvl], _CIN[lvl], _COUT[lvl]))
            g += 1

    n_steps = g
    copies[0].start()
    for g in range(n_steps):
        if g + 1 < n_steps:
            copies[g + 1].start()
        copies[g].wait()
        computes[g]()


def _mega(x0, Ub0, W0, b0, Ub1, W1, b1, Ub2, W2, b2, U0, L0, U1, L1, U2, L2):
    f32 = jnp.float32
    vmem = pl.BlockSpec(memory_space=pltpu.MemorySpace.VMEM)
    hbm = pl.BlockSpec(memory_space=pltpu.MemorySpace.HBM)
    return pl.pallas_call(
        _mega_body,
        in_specs=[vmem] * 10 + [hbm] * 6,
        out_specs=vmem,
        out_shape=jax.ShapeDtypeStruct((_NODES[2], _B * _COUT[2]), f32),
        scratch_shapes=[
            pltpu.VMEM((2, _TM, 512), f32),
            pltpu.VMEM((2, _TM, 1024), f32),
            pltpu.VMEM((2, _TM, 2048), f32),
            pltpu.VMEM((2, _TM, 4096), f32),
            pltpu.VMEM((_NODES[0], _B * _CIN[0]), f32),
            pltpu.VMEM((_NODES[0], _B * _COUT[0]), f32),
            pltpu.VMEM((_NODES[1], _B * _COUT[0]), f32),
            pltpu.VMEM((_NODES[1], _B * _COUT[1]), f32),
            pltpu.VMEM((_NODES[2], _B * _COUT[1]), f32),
            pltpu.SemaphoreType.DMA((2,)),
        ],
    )(x0, Ub0, W0, b0.reshape(1, -1), Ub1, W1, b1.reshape(1, -1),
      Ub2, W2, b2.reshape(1, -1), U0, L0, U1, L1, U2, L2)


def kernel(latents, fc_W, fc_b, U0, Ub0, L0, W0, b0, U1, Ub1, L1, W1, b1, U2, Ub2, L2, W2, b2):
    feats = _fc(latents, fc_W, fc_b)
    x0 = feats.reshape(_B, _N0, _CHANNELS[0]).transpose(1, 0, 2).reshape(_N0, _B * _CHANNELS[0])
    out = _mega(x0, Ub0, W0, b0, Ub1, W1, b1, Ub2, W2, b2, U0, L0, U1, L1, U2, L2)
    return out.reshape(_NODES[2], _B, _COUT[2]).transpose(1, 0, 2)
